# Initial kernel scaffold; baseline (speedup 1.0000x reference)
import functools

import jax
import jax.numpy as jnp
from jax import lax
from jax.experimental import pallas as pl
from jax.experimental.pallas import tpu as pltpu
from jax.experimental.pallas import tpu_sc as plsc

F32 = jnp.float32
I32 = jnp.int32

NSC = 2
NTILE = 16
CH = 128
RB = 400


def _make_sc_kernels(N, E, HALF):
    RPT = -(-(N + 1) // NTILE)
    NPAD = RPT * NTILE
    EPAD = -(-E // (NSC * NTILE * CH)) * (NSC * NTILE * CH)
    EPT1 = EPAD // NTILE
    NCH1 = EPT1 // CH
    EPT2 = EPAD // (NSC * NTILE)
    NCH2 = EPT2 // CH

    mesh = plsc.VectorSubcoreMesh(core_axis_name="c", subcore_axis_name="s")

    @functools.partial(
        pl.kernel, mesh=mesh,
        out_type=jax.ShapeDtypeStruct((NSC * NPAD, 16), F32),
        scratch_types=[
            pltpu.VMEM((CH,), I32),
            pltpu.VMEM((CH, 16), F32),
            pltpu.VMEM_SHARED((NPAD, 16), F32),
        ],
    )
    def deg_kernel(dst_hbm, ones_hbm, zeros_hbm, out_hbm, dstv, onesv, dacc):
        c = lax.axis_index("c")
        s = lax.axis_index("s")
        wid = s * NSC + c
        pltpu.sync_copy(zeros_hbm, dacc.at[pl.ds(s * RPT, RPT)])
        pltpu.sync_copy(ones_hbm, onesv)
        plsc.subcore_barrier()

        def body(j, carry):
            base = wid * EPT2 + j * CH
            pltpu.sync_copy(dst_hbm.at[pl.ds(base, CH)], dstv)
            pltpu.sync_copy(onesv, dacc.at[dstv], add=True)
            return carry

        lax.fori_loop(0, NCH2, body, 0)
        plsc.subcore_barrier()
        pltpu.sync_copy(dacc.at[pl.ds(s * RPT, RPT)],
                        out_hbm.at[pl.ds(c * NPAD + s * RPT, RPT)])

    @functools.partial(
        pl.kernel, mesh=mesh,
        out_type=[jax.ShapeDtypeStruct((NPAD, HALF), F32)] * 2,
        scratch_types=[
            pltpu.VMEM((CH,), I32),
            pltpu.VMEM((CH,), I32),
            pltpu.VMEM((CH, HALF), F32),
            pltpu.VMEM_SHARED((NPAD, HALF), F32),
            pltpu.SemaphoreType.DMA,
        ],
    )
    def seg1_kernel(glo_hbm, ghi_hbm, src_hbm, dst_hbm, zeros_hbm,
                    olo_hbm, ohi_hbm, srcv, dstv, rowsv, acc, sem):
        c = lax.axis_index("c")
        s = lax.axis_index("s")
        pltpu.sync_copy(zeros_hbm, acc.at[pl.ds(s * RPT, RPT)])
        plsc.subcore_barrier()

        def body(j, carry):
            base = s * EPT1 + j * CH
            pltpu.sync_copy(src_hbm.at[pl.ds(base, CH)], srcv)
            pltpu.sync_copy(dst_hbm.at[pl.ds(base, CH)], dstv)

            @pl.when(c == 0)
            def _():
                pltpu.async_copy(glo_hbm.at[srcv], rowsv, sem).wait()

            @pl.when(c == 1)
            def _():
                pltpu.async_copy(ghi_hbm.at[srcv], rowsv, sem).wait()

            pltpu.sync_copy(rowsv, acc.at[dstv], add=True)
            return carry

        lax.fori_loop(0, NCH1, body, 0)
        plsc.subcore_barrier()

        @pl.when(c == 0)
        def _():
            pltpu.sync_copy(acc.at[pl.ds(s * RPT, RPT)],
                            olo_hbm.at[pl.ds(s * RPT, RPT)])

        @pl.when(c == 1)
        def _():
            pltpu.sync_copy(acc.at[pl.ds(s * RPT, RPT)],
                            ohi_hbm.at[pl.ds(s * RPT, RPT)])

    @functools.partial(
        pl.kernel, mesh=mesh,
        out_type=jax.ShapeDtypeStruct((NSC * NPAD, HALF), F32),
        scratch_types=[
            pltpu.VMEM((CH,), I32),
            pltpu.VMEM((CH,), I32),
            pltpu.VMEM((CH, HALF), F32),
            pltpu.VMEM_SHARED((NPAD, HALF), F32),
            pltpu.SemaphoreType.DMA,
        ],
    )
    def seg2_kernel(g_hbm, src_hbm, dst_hbm, zeros_hbm, out_hbm,
                    srcv, dstv, rowsv, acc, sem):
        c = lax.axis_index("c")
        s = lax.axis_index("s")
        wid = s * NSC + c
        pltpu.sync_copy(zeros_hbm, acc.at[pl.ds(s * RPT, RPT)])
        plsc.subcore_barrier()

        def body(j, carry):
            base = wid * EPT2 + j * CH
            pltpu.sync_copy(src_hbm.at[pl.ds(base, CH)], srcv)
            pltpu.sync_copy(dst_hbm.at[pl.ds(base, CH)], dstv)
            pltpu.async_copy(g_hbm.at[srcv], rowsv, sem).wait()
            pltpu.sync_copy(rowsv, acc.at[dstv], add=True)
            return carry

        lax.fori_loop(0, NCH2, body, 0)
        plsc.subcore_barrier()
        pltpu.sync_copy(acc.at[pl.ds(s * RPT, RPT)],
                        out_hbm.at[pl.ds(c * NPAD + s * RPT, RPT)])

    return deg_kernel, seg1_kernel, seg2_kernel, NPAD, RPT, EPAD


def kernel(x, edge_index, W1, b1, W2, b2):
    N, d_in = x.shape
    E = edge_index.shape[1]
    d_hid = W1.shape[1]
    d_out = W2.shape[1]
    HALF = d_hid // 2

    deg_kernel, seg1_kernel, seg2_kernel, NPAD, RPT, EPAD = _make_sc_kernels(
        N, E, HALF)

    G = N // RB
    OFF = NPAD // RB

    src = edge_index[0]
    dst = edge_index[1]
    pad = EPAD - E
    src_p = jnp.concatenate([src, jnp.zeros((pad,), I32)])
    dst_p = jnp.concatenate([dst, jnp.full((pad,), N, I32)])
    ones16 = jnp.ones((CH, 16), F32)
    zeros16 = jnp.zeros((RPT, 16), F32)
    zerosH = jnp.zeros((RPT, HALF), F32)
    b1r = b1.reshape(1, d_hid)
    b2r = b2.reshape(1, d_out)

    deg2 = deg_kernel(dst_p, ones16, zeros16)

    def tc1(dg0, dg1, xr, w1, glo, ghi):
        deg = dg0[:, 0:1] + dg1[:, 0:1] + 1.0
        dinv = lax.rsqrt(deg)
        g = jnp.dot(xr[...], w1[...], preferred_element_type=F32) * dinv
        glo[...] = g[:, :HALF]
        ghi[...] = g[:, HALF:]

    glo, ghi = pl.pallas_call(
        tc1,
        grid=(G,),
        in_specs=[
            pl.BlockSpec((RB, 16), lambda i: (i, 0)),
            pl.BlockSpec((RB, 16), lambda i: (i + OFF, 0)),
            pl.BlockSpec((RB, d_in), lambda i: (i, 0)),
            pl.BlockSpec((d_in, d_hid), lambda i: (0, 0)),
        ],
        out_specs=[pl.BlockSpec((RB, HALF), lambda i: (i, 0))] * 2,
        out_shape=[jax.ShapeDtypeStruct((N, HALF), F32)] * 2,
    )(deg2, deg2, x, W1)

    alo, ahi = seg1_kernel(glo, ghi, src_p, dst_p, zerosH)

    def tc3(dg0, dg1, al, ah, gl, gh, b1_, w2, out):
        deg = dg0[:, 0:1] + dg1[:, 0:1] + 1.0
        dinv = lax.rsqrt(deg)
        z = jnp.concatenate([al[...] + gl[...], ah[...] + gh[...]], axis=1)
        z = jnp.maximum(z * dinv + b1_[...], 0.0)
        out[...] = jnp.dot(z, w2[...], preferred_element_type=F32) * dinv

    g2 = pl.pallas_call(
        tc3,
        grid=(G,),
        in_specs=[
            pl.BlockSpec((RB, 16), lambda i: (i, 0)),
            pl.BlockSpec((RB, 16), lambda i: (i + OFF, 0)),
            pl.BlockSpec((RB, HALF), lambda i: (i, 0)),
            pl.BlockSpec((RB, HALF), lambda i: (i, 0)),
            pl.BlockSpec((RB, HALF), lambda i: (i, 0)),
            pl.BlockSpec((RB, HALF), lambda i: (i, 0)),
            pl.BlockSpec((1, d_hid), lambda i: (0, 0)),
            pl.BlockSpec((d_hid, d_out), lambda i: (0, 0)),
        ],
        out_specs=pl.BlockSpec((RB, d_out), lambda i: (i, 0)),
        out_shape=jax.ShapeDtypeStruct((N, d_out), F32),
    )(deg2, deg2, alo, ahi, glo, ghi, b1r, W2)

    a2 = seg2_kernel(g2, src_p, dst_p, zerosH)

    def tc5(dg0, dg1, p0, p1, gg, b2_, out):
        deg = dg0[:, 0:1] + dg1[:, 0:1] + 1.0
        dinv = lax.rsqrt(deg)
        acc = p0[...] + p1[...] + gg[...]
        out[...] = jnp.maximum(acc * dinv + b2_[...], 0.0)

    out = pl.pallas_call(
        tc5,
        grid=(G,),
        in_specs=[
            pl.BlockSpec((RB, 16), lambda i: (i, 0)),
            pl.BlockSpec((RB, 16), lambda i: (i + OFF, 0)),
            pl.BlockSpec((RB, d_out), lambda i: (i, 0)),
            pl.BlockSpec((RB, d_out), lambda i: (i + OFF, 0)),
            pl.BlockSpec((RB, d_out), lambda i: (i, 0)),
            pl.BlockSpec((1, d_out), lambda i: (0, 0)),
        ],
        out_specs=pl.BlockSpec((RB, d_out), lambda i: (i, 0)),
        out_shape=jax.ShapeDtypeStruct((N, d_out), F32),
    )(deg2, deg2, a2, a2, g2, b2r)

    return out


# speedup vs baseline: 9.6046x; 9.6046x over previous
import functools

import jax
import jax.numpy as jnp
from jax import lax
from jax.experimental import pallas as pl
from jax.experimental.pallas import tpu as pltpu
from jax.experimental.pallas import tpu_sc as plsc

F32 = jnp.float32
I32 = jnp.int32

NSC = 2
NTILE = 16
CH = 128
RB = 400


def _make_sc_kernels(N, E, HALF):
    NPAD = -(-(N + 1) // 3200) * 3200
    RPT = NPAD // NTILE
    EPAD = -(-E // (NSC * NTILE * CH)) * (NSC * NTILE * CH)
    EPT1 = EPAD // NTILE
    NCH1 = EPT1 // CH
    EPT2 = EPAD // (NSC * NTILE)
    NCH2 = EPT2 // CH

    mesh = plsc.VectorSubcoreMesh(core_axis_name="c", subcore_axis_name="s")

    @functools.partial(
        pl.kernel, mesh=mesh,
        out_type=jax.ShapeDtypeStruct((NSC * NPAD, 16), F32),
        scratch_types=[
            pltpu.VMEM((CH,), I32),
            pltpu.VMEM((CH, 16), F32),
            pltpu.VMEM_SHARED((NPAD, 16), F32),
        ],
    )
    def deg_kernel(dst_hbm, ones_hbm, zeros_hbm, out_hbm, dstv, onesv, dacc):
        c = lax.axis_index("c")
        s = lax.axis_index("s")
        wid = s * NSC + c
        pltpu.sync_copy(zeros_hbm, dacc.at[pl.ds(s * RPT, RPT)])
        pltpu.sync_copy(ones_hbm, onesv)
        plsc.subcore_barrier()

        def body(j, carry):
            base = wid * EPT2 + j * CH
            pltpu.sync_copy(dst_hbm.at[pl.ds(base, CH)], dstv)
            pltpu.sync_copy(onesv, dacc.at[dstv], add=True)
            return carry

        lax.fori_loop(0, NCH2, body, 0)
        plsc.subcore_barrier()
        pltpu.sync_copy(dacc.at[pl.ds(s * RPT, RPT)],
                        out_hbm.at[pl.ds(c * NPAD + s * RPT, RPT)])

    @functools.partial(
        pl.kernel, mesh=mesh,
        out_type=[jax.ShapeDtypeStruct((NPAD, HALF), F32)] * 2,
        scratch_types=[
            pltpu.VMEM((CH,), I32),
            pltpu.VMEM((CH,), I32),
            pltpu.VMEM((CH, HALF), F32),
            pltpu.VMEM_SHARED((NPAD, HALF), F32),
            pltpu.SemaphoreType.DMA,
        ],
    )
    def seg1_kernel(glo_hbm, ghi_hbm, src_hbm, dst_hbm, zeros_hbm,
                    olo_hbm, ohi_hbm, srcv, dstv, rowsv, acc, sem):
        c = lax.axis_index("c")
        s = lax.axis_index("s")
        pltpu.sync_copy(zeros_hbm, acc.at[pl.ds(s * RPT, RPT)])
        plsc.subcore_barrier()

        def body(j, carry):
            base = s * EPT1 + j * CH
            pltpu.sync_copy(src_hbm.at[pl.ds(base, CH)], srcv)
            pltpu.sync_copy(dst_hbm.at[pl.ds(base, CH)], dstv)

            @pl.when(c == 0)
            def _():
                pltpu.async_copy(glo_hbm.at[srcv], rowsv, sem).wait()

            @pl.when(c == 1)
            def _():
                pltpu.async_copy(ghi_hbm.at[srcv], rowsv, sem).wait()

            pltpu.sync_copy(rowsv, acc.at[dstv], add=True)
            return carry

        lax.fori_loop(0, NCH1, body, 0)
        plsc.subcore_barrier()

        @pl.when(c == 0)
        def _():
            pltpu.sync_copy(acc.at[pl.ds(s * RPT, RPT)],
                            olo_hbm.at[pl.ds(s * RPT, RPT)])

        @pl.when(c == 1)
        def _():
            pltpu.sync_copy(acc.at[pl.ds(s * RPT, RPT)],
                            ohi_hbm.at[pl.ds(s * RPT, RPT)])

    @functools.partial(
        pl.kernel, mesh=mesh,
        out_type=jax.ShapeDtypeStruct((NSC * NPAD, HALF), F32),
        scratch_types=[
            pltpu.VMEM((CH,), I32),
            pltpu.VMEM((CH,), I32),
            pltpu.VMEM((CH, HALF), F32),
            pltpu.VMEM_SHARED((NPAD, HALF), F32),
            pltpu.SemaphoreType.DMA,
        ],
    )
    def seg2_kernel(g_hbm, src_hbm, dst_hbm, zeros_hbm, out_hbm,
                    srcv, dstv, rowsv, acc, sem):
        c = lax.axis_index("c")
        s = lax.axis_index("s")
        wid = s * NSC + c
        pltpu.sync_copy(zeros_hbm, acc.at[pl.ds(s * RPT, RPT)])
        plsc.subcore_barrier()

        def body(j, carry):
            base = wid * EPT2 + j * CH
            pltpu.sync_copy(src_hbm.at[pl.ds(base, CH)], srcv)
            pltpu.sync_copy(dst_hbm.at[pl.ds(base, CH)], dstv)
            pltpu.async_copy(g_hbm.at[srcv], rowsv, sem).wait()
            pltpu.sync_copy(rowsv, acc.at[dstv], add=True)
            return carry

        lax.fori_loop(0, NCH2, body, 0)
        plsc.subcore_barrier()
        pltpu.sync_copy(acc.at[pl.ds(s * RPT, RPT)],
                        out_hbm.at[pl.ds(c * NPAD + s * RPT, RPT)])

    return deg_kernel, seg1_kernel, seg2_kernel, NPAD, RPT, EPAD


def kernel(x, edge_index, W1, b1, W2, b2):
    N, d_in = x.shape
    E = edge_index.shape[1]
    d_hid = W1.shape[1]
    d_out = W2.shape[1]
    HALF = d_hid // 2

    deg_kernel, seg1_kernel, seg2_kernel, NPAD, RPT, EPAD = _make_sc_kernels(
        N, E, HALF)

    G = N // RB
    OFF = NPAD // RB

    src = edge_index[0]
    dst = edge_index[1]
    pad = EPAD - E
    src_p = jnp.concatenate([src, jnp.zeros((pad,), I32)])
    dst_p = jnp.concatenate([dst, jnp.full((pad,), N, I32)])
    ones16 = jnp.ones((CH, 16), F32)
    zeros16 = jnp.zeros((RPT, 16), F32)
    zerosH = jnp.zeros((RPT, HALF), F32)
    b1r = b1.reshape(1, d_hid)
    b2r = b2.reshape(1, d_out)

    deg2 = deg_kernel(dst_p, ones16, zeros16)

    def tc1(dg0, dg1, xr, w1, glo, ghi):
        deg = dg0[:, 0:1] + dg1[:, 0:1] + 1.0
        dinv = lax.rsqrt(deg)
        g = jnp.dot(xr[...], w1[...], preferred_element_type=F32) * dinv
        glo[...] = g[:, :HALF]
        ghi[...] = g[:, HALF:]

    glo, ghi = pl.pallas_call(
        tc1,
        grid=(G,),
        in_specs=[
            pl.BlockSpec((RB, 16), lambda i: (i, 0)),
            pl.BlockSpec((RB, 16), lambda i: (i + OFF, 0)),
            pl.BlockSpec((RB, d_in), lambda i: (i, 0)),
            pl.BlockSpec((d_in, d_hid), lambda i: (0, 0)),
        ],
        out_specs=[pl.BlockSpec((RB, HALF), lambda i: (i, 0))] * 2,
        out_shape=[jax.ShapeDtypeStruct((N, HALF), F32)] * 2,
    )(deg2, deg2, x, W1)

    alo, ahi = seg1_kernel(glo, ghi, src_p, dst_p, zerosH)

    def tc3(dg0, dg1, al, ah, gl, gh, b1_, w2, out):
        deg = dg0[:, 0:1] + dg1[:, 0:1] + 1.0
        dinv = lax.rsqrt(deg)
        z = jnp.concatenate([al[...] + gl[...], ah[...] + gh[...]], axis=1)
        z = jnp.maximum(z * dinv + b1_[...], 0.0)
        out[...] = jnp.dot(z, w2[...], preferred_element_type=F32) * dinv

    g2 = pl.pallas_call(
        tc3,
        grid=(G,),
        in_specs=[
            pl.BlockSpec((RB, 16), lambda i: (i, 0)),
            pl.BlockSpec((RB, 16), lambda i: (i + OFF, 0)),
            pl.BlockSpec((RB, HALF), lambda i: (i, 0)),
            pl.BlockSpec((RB, HALF), lambda i: (i, 0)),
            pl.BlockSpec((RB, HALF), lambda i: (i, 0)),
            pl.BlockSpec((RB, HALF), lambda i: (i, 0)),
            pl.BlockSpec((1, d_hid), lambda i: (0, 0)),
            pl.BlockSpec((d_hid, d_out), lambda i: (0, 0)),
        ],
        out_specs=pl.BlockSpec((RB, d_out), lambda i: (i, 0)),
        out_shape=jax.ShapeDtypeStruct((N, d_out), F32),
    )(deg2, deg2, alo, ahi, glo, ghi, b1r, W2)

    a2 = seg2_kernel(g2, src_p, dst_p, zerosH)

    def tc5(dg0, dg1, p0, p1, gg, b2_, out):
        deg = dg0[:, 0:1] + dg1[:, 0:1] + 1.0
        dinv = lax.rsqrt(deg)
        acc = p0[...] + p1[...] + gg[...]
        out[...] = jnp.maximum(acc * dinv + b2_[...], 0.0)

    out = pl.pallas_call(
        tc5,
        grid=(G,),
        in_specs=[
            pl.BlockSpec((RB, 16), lambda i: (i, 0)),
            pl.BlockSpec((RB, 16), lambda i: (i + OFF, 0)),
            pl.BlockSpec((RB, d_out), lambda i: (i, 0)),
            pl.BlockSpec((RB, d_out), lambda i: (i + OFF, 0)),
            pl.BlockSpec((RB, d_out), lambda i: (i, 0)),
            pl.BlockSpec((1, d_out), lambda i: (0, 0)),
        ],
        out_specs=pl.BlockSpec((RB, d_out), lambda i: (i, 0)),
        out_shape=jax.ShapeDtypeStruct((N, d_out), F32),
    )(deg2, deg2, a2, a2, g2, b2r)

    return out
